# baseline (speedup 1.0000x reference)
import functools

import jax
import jax.numpy as jnp
from jax import lax
from jax.experimental import pallas as pl
from jax.experimental.pallas import tpu as pltpu
from jax.experimental.pallas import tpu_sc as plsc

NC = 2
NS = 16
NW = NC * NS
CH = 128
NB = 4
NR = 8
DH = 32


def _load_index_chunks(ei3_hbm, which, idx_v, wid, k, nchunks):
    last_full = nchunks // k
    klast = nchunks - last_full * k

    @pl.when(wid < last_full)
    def _():
        pltpu.sync_copy(ei3_hbm.at[which, pl.ds(wid * k, k)], idx_v)

    if klast > 0:
        @pl.when(wid == last_full)
        def _():
            pltpu.sync_copy(ei3_hbm.at[which, pl.ds(last_full * k, klast)],
                            idx_v.at[pl.ds(0, klast)])


def _sc_degree(ei3, zeros32, ones32, nt, k):
    nchunks = ei3.shape[1]
    rpt = nt // NS
    mesh = plsc.VectorSubcoreMesh(core_axis_name="c", subcore_axis_name="s")

    @functools.partial(
        pl.kernel,
        out_type=jax.ShapeDtypeStruct((NC, nt, DH), jnp.float32),
        mesh=mesh,
        scratch_types=[
            pltpu.VMEM((k, CH), jnp.int32),
            pltpu.VMEM((CH, DH), jnp.float32),
            pltpu.VMEM_SHARED((nt, DH), jnp.float32),
        ],
        compiler_params=pltpu.CompilerParams(use_tc_tiling_on_sc=False),
    )
    def deg_kernel(ei3_hbm, z_hbm, ones_hbm, out_hbm, idx_v, ones_v, acc_sh):
        c = lax.axis_index("c")
        s = lax.axis_index("s")
        wid = c * NS + s
        kw = jnp.clip(nchunks - wid * k, 0, k)
        pltpu.sync_copy(ones_hbm, ones_v)
        pltpu.sync_copy(z_hbm, acc_sh.at[pl.ds(s * rpt, rpt)])
        _load_index_chunks(ei3_hbm, 1, idx_v, wid, k, nchunks)
        plsc.subcore_barrier()

        def step(j, carry):
            @pl.when(j < kw)
            def _():
                pltpu.sync_copy(ones_v, acc_sh.at[idx_v.at[j]], add=True)

            return carry

        lax.fori_loop(0, k, step, 0)
        plsc.subcore_barrier()
        pltpu.sync_copy(acc_sh.at[pl.ds(s * rpt, rpt)],
                        out_hbm.at[c, pl.ds(s * rpt, rpt)])

    return deg_kernel(ei3, zeros32, ones32).reshape(NC, nt // 4, CH)


def _sc_scatter(gp, ei3, zeros32, nt, k):
    g = gp.reshape(nt, DH)
    nchunks = ei3.shape[1]
    rpt = nt // NS
    mesh = plsc.VectorSubcoreMesh(core_axis_name="c", subcore_axis_name="s")

    @functools.partial(
        pl.kernel,
        out_type=jax.ShapeDtypeStruct((NC, nt, DH), jnp.bfloat16),
        mesh=mesh,
        scratch_types=[
            pltpu.VMEM((k, CH), jnp.int32),
            pltpu.VMEM((k, CH), jnp.int32),
            pltpu.VMEM((NR, CH, DH), jnp.bfloat16),
            pltpu.VMEM_SHARED((nt, DH), jnp.bfloat16),
            pltpu.SemaphoreType.DMA((NR,)),
            pltpu.SemaphoreType.DMA((NR,)),
        ],
        compiler_params=pltpu.CompilerParams(use_tc_tiling_on_sc=False),
    )
    def scat_kernel(g_hbm, ei3_hbm, z_hbm, out_hbm,
                    isrc_v, idst_v, rows_v, acc_sh, gsems, ssems):
        c = lax.axis_index("c")
        s = lax.axis_index("s")
        wid = c * NS + s
        kw = jnp.clip(nchunks - wid * k, 0, k)
        pltpu.sync_copy(z_hbm, acc_sh.at[pl.ds(s * rpt, rpt)])
        _load_index_chunks(ei3_hbm, 0, isrc_v, wid, k, nchunks)
        _load_index_chunks(ei3_hbm, 1, idst_v, wid, k, nchunks)
        plsc.subcore_barrier()

        def gather(j, b):
            pltpu.async_copy(g_hbm.at[isrc_v.at[j]], rows_v.at[b], gsems.at[b])

        def gather_wait(j, b):
            pltpu.make_async_copy(g_hbm.at[isrc_v.at[j]], rows_v.at[b],
                                  gsems.at[b]).wait()

        def scat(j, b):
            pltpu.async_copy(rows_v.at[b], acc_sh.at[idst_v.at[j]], ssems.at[b],
                             add=True)

        def scat_wait(j, b):
            pltpu.make_async_copy(rows_v.at[b], acc_sh.at[idst_v.at[j]],
                                  ssems.at[b]).wait()

        for b in range(min(NB, k)):
            @pl.when(b < kw)
            def _():
                gather(b, b)

        def step(j, carry):
            b = lax.rem(j, NR)

            @pl.when(j < kw)
            def _():
                gather_wait(j, b)
                scat(j, b)
                jn = j + NB
                bn = lax.rem(jn, NR)

                @pl.when(jn < kw)
                def _():
                    @pl.when(jn >= NR)
                    def _():
                        scat_wait(jn - NR, bn)
                    gather(jn, bn)

            return carry

        lax.fori_loop(0, k, step, 0)

        def drain(j2, carry):
            jj = kw - NR + j2

            @pl.when(jj >= jnp.maximum(kw - NR, 0))
            def _():
                scat_wait(jj, lax.rem(jj, NR))

            return carry

        lax.fori_loop(jnp.maximum(NR - kw, 0), NR, drain, 0)
        plsc.subcore_barrier()
        pltpu.sync_copy(acc_sh.at[pl.ds(s * rpt, rpt)],
                        out_hbm.at[c, pl.ds(s * rpt, rpt)])

    return scat_kernel(g, ei3, zeros32).reshape(NC, nt // 4, CH)


def _tc_h(x4, W1s, nt, r):
    rp = r // 4

    def body(x_ref, w_ref, o_ref):
        o_ref[...] = jnp.dot(x_ref[...], w_ref[...],
                             preferred_element_type=jnp.float32)

    return pl.pallas_call(
        body,
        grid=(nt // r,),
        in_specs=[
            pl.BlockSpec((rp, x4.shape[1]), lambda i: (i, 0)),
            pl.BlockSpec(W1s.shape, lambda i: (0, 0)),
        ],
        out_specs=pl.BlockSpec((rp, CH), lambda i: (i, 0)),
        out_shape=jax.ShapeDtypeStruct((nt // 4, CH), jnp.float32),
    )(x4, W1s)


def _tc_scale(hp, degp, nt, r):
    rp = r // 4

    def body(h_ref, d_ref, o_ref):
        dinv = lax.rsqrt(d_ref[0] + d_ref[1] + 1.0)
        o_ref[...] = (h_ref[...] * dinv).astype(jnp.bfloat16)

    return pl.pallas_call(
        body,
        grid=(nt // r,),
        in_specs=[
            pl.BlockSpec((rp, CH), lambda i: (i, 0)),
            pl.BlockSpec((NC, rp, CH), lambda i: (0, i, 0)),
        ],
        out_specs=pl.BlockSpec((rp, CH), lambda i: (i, 0)),
        out_shape=jax.ShapeDtypeStruct((nt // 4, CH), jnp.bfloat16),
    )(hp, degp)


def _tc_mid(sp, gp, degp, b1p, W2s, nt, r):
    rp = r // 4

    def body(s_ref, g_ref, d_ref, b_ref, w_ref, o_ref):
        dinv = lax.rsqrt(d_ref[0] + d_ref[1] + 1.0)
        stot = (s_ref[0] + s_ref[1] + g_ref[...]).astype(jnp.float32)
        z = jnp.maximum(stot * dinv + b_ref[...], 0.0)
        h = jnp.dot(z, w_ref[...], preferred_element_type=jnp.float32)
        o_ref[...] = (h * dinv).astype(jnp.bfloat16)

    return pl.pallas_call(
        body,
        grid=(nt // r,),
        in_specs=[
            pl.BlockSpec((NC, rp, CH), lambda i: (0, i, 0)),
            pl.BlockSpec((rp, CH), lambda i: (i, 0)),
            pl.BlockSpec((NC, rp, CH), lambda i: (0, i, 0)),
            pl.BlockSpec(b1p.shape, lambda i: (0, 0)),
            pl.BlockSpec(W2s.shape, lambda i: (0, 0)),
        ],
        out_specs=pl.BlockSpec((rp, CH), lambda i: (i, 0)),
        out_shape=jax.ShapeDtypeStruct((nt // 4, CH), jnp.bfloat16),
    )(sp, gp, degp, b1p, W2s)


def _tc_head(sp, gp, degp, b2p, Wfs, bfp, nt, r):
    rp = r // 4

    def body(s_ref, g_ref, d_ref, b_ref, w_ref, bf_ref, o_ref):
        dinv = lax.rsqrt(d_ref[0] + d_ref[1] + 1.0)
        stot = (s_ref[0] + s_ref[1] + g_ref[...]).astype(jnp.float32)
        z = jnp.maximum(stot * dinv + b_ref[...], 0.0)
        h = jnp.dot(z, w_ref[...], preferred_element_type=jnp.float32)
        o_ref[...] = jax.nn.sigmoid(h + bf_ref[...])

    return pl.pallas_call(
        body,
        grid=(nt // r,),
        in_specs=[
            pl.BlockSpec((NC, rp, CH), lambda i: (0, i, 0)),
            pl.BlockSpec((rp, CH), lambda i: (i, 0)),
            pl.BlockSpec((NC, rp, CH), lambda i: (0, i, 0)),
            pl.BlockSpec(b2p.shape, lambda i: (0, 0)),
            pl.BlockSpec(Wfs.shape, lambda i: (0, 0)),
            pl.BlockSpec(bfp.shape, lambda i: (0, 0)),
        ],
        out_specs=pl.BlockSpec((rp, Wfs.shape[1]), lambda i: (i, 0)),
        out_shape=jax.ShapeDtypeStruct((nt // 4, Wfs.shape[1]), jnp.float32),
    )(sp, gp, degp, b2p, Wfs, bfp)


def kernel(x, edge_index, W1, b1, W2, b2, Wfc, bfc):
    n, din = x.shape
    e = edge_index.shape[1]
    r = 2048
    nt = -(-n // r) * r
    dout = Wfc.shape[1]
    assert e % CH == 0 and nt % (8 * NS) == 0

    ei3 = edge_index.reshape(2, e // CH, CH)
    k = -(-(e // CH) // NW)

    x4 = jnp.pad(x, ((0, nt - n), (0, 0))).reshape(nt // 4, 4 * din)
    zeros32 = jnp.zeros((nt // NS, DH), jnp.float32)
    zerosb = jnp.zeros((nt // NS, DH), jnp.bfloat16)
    ones32 = jnp.ones((CH, DH), jnp.float32)

    eye4 = jnp.eye(4, dtype=jnp.float32)
    W1s = jnp.kron(eye4, W1)
    W2s = jnp.kron(eye4, W2)
    wfc_p = jnp.pad(Wfc, ((0, 0), (0, 8 - dout)))
    Wfs = jnp.kron(eye4, wfc_p)
    b1p = jnp.tile(b1, 4).reshape(1, CH)
    b2p = jnp.tile(b2, 4).reshape(1, CH)
    bfp = jnp.tile(jnp.pad(bfc, (0, 8 - dout)), 4).reshape(1, DH)

    degp = _sc_degree(ei3, zeros32, ones32, nt, k)
    hp = _tc_h(x4, W1s, nt, r)
    g1p = _tc_scale(hp, degp, nt, r)
    s1p = _sc_scatter(g1p, ei3, zerosb, nt, k)
    g2p = _tc_mid(s1p, g1p, degp, b1p, W2s, nt, r)
    s2p = _sc_scatter(g2p, ei3, zerosb, nt, k)
    outp = _tc_head(s2p, g2p, degp, b2p, Wfs, bfp, nt, r)

    return outp[:n // 4].reshape(n, 8)[:, :dout]


# speedup vs baseline: 1.0326x; 1.0326x over previous
import functools

import jax
import jax.numpy as jnp
from jax import lax
from jax.experimental import pallas as pl
from jax.experimental.pallas import tpu as pltpu
from jax.experimental.pallas import tpu_sc as plsc

NC = 2
NS = 16
NW = NC * NS
CH = 128
NB = 4
NR = 8
DH = 32


def _load_index_chunks(ei3_hbm, which, idx_v, wid, k, nchunks):
    last_full = nchunks // k
    klast = nchunks - last_full * k

    @pl.when(wid < last_full)
    def _():
        pltpu.sync_copy(ei3_hbm.at[which, pl.ds(wid * k, k)], idx_v)

    if klast > 0:
        @pl.when(wid == last_full)
        def _():
            pltpu.sync_copy(ei3_hbm.at[which, pl.ds(last_full * k, klast)],
                            idx_v.at[pl.ds(0, klast)])


def _sc_degree(ei3, zeros16, ones16, nt, k):
    nchunks = ei3.shape[1]
    rpt = nt // NS
    mesh = plsc.VectorSubcoreMesh(core_axis_name="c", subcore_axis_name="s")

    @functools.partial(
        pl.kernel,
        out_type=jax.ShapeDtypeStruct((NC, nt, 16), jnp.float32),
        mesh=mesh,
        scratch_types=[
            pltpu.VMEM((k, CH), jnp.int32),
            pltpu.VMEM((CH, 16), jnp.float32),
            pltpu.VMEM_SHARED((nt, 16), jnp.float32),
        ],
        compiler_params=pltpu.CompilerParams(use_tc_tiling_on_sc=False),
    )
    def deg_kernel(ei3_hbm, z_hbm, ones_hbm, out_hbm, idx_v, ones_v, acc_sh):
        c = lax.axis_index("c")
        s = lax.axis_index("s")
        wid = c * NS + s
        kw = jnp.clip(nchunks - wid * k, 0, k)
        pltpu.sync_copy(ones_hbm, ones_v)
        pltpu.sync_copy(z_hbm, acc_sh.at[pl.ds(s * rpt, rpt)])
        _load_index_chunks(ei3_hbm, 1, idx_v, wid, k, nchunks)
        plsc.subcore_barrier()

        def step(j, carry):
            @pl.when(j < kw)
            def _():
                pltpu.sync_copy(ones_v, acc_sh.at[idx_v.at[j]], add=True)

            return carry

        lax.fori_loop(0, k, step, 0)
        plsc.subcore_barrier()
        pltpu.sync_copy(acc_sh.at[pl.ds(s * rpt, rpt)],
                        out_hbm.at[c, pl.ds(s * rpt, rpt)])

    return deg_kernel(ei3, zeros16, ones16).reshape(NC, nt // 8, CH)


def _sc_scatter(gp, ei3, zeros32, nt, k):
    g = gp.reshape(nt, DH)
    nchunks = ei3.shape[1]
    rpt = nt // NS
    mesh = plsc.VectorSubcoreMesh(core_axis_name="c", subcore_axis_name="s")

    @functools.partial(
        pl.kernel,
        out_type=jax.ShapeDtypeStruct((NC, nt, DH), jnp.bfloat16),
        mesh=mesh,
        scratch_types=[
            pltpu.VMEM((k, CH), jnp.int32),
            pltpu.VMEM((k, CH), jnp.int32),
            pltpu.VMEM((NR, CH, DH), jnp.bfloat16),
            pltpu.VMEM_SHARED((nt, DH), jnp.bfloat16),
            pltpu.SemaphoreType.DMA((NR,)),
            pltpu.SemaphoreType.DMA((NR,)),
        ],
        compiler_params=pltpu.CompilerParams(use_tc_tiling_on_sc=False),
    )
    def scat_kernel(g_hbm, ei3_hbm, z_hbm, out_hbm,
                    isrc_v, idst_v, rows_v, acc_sh, gsems, ssems):
        c = lax.axis_index("c")
        s = lax.axis_index("s")
        wid = c * NS + s
        kw = jnp.clip(nchunks - wid * k, 0, k)
        pltpu.sync_copy(z_hbm, acc_sh.at[pl.ds(s * rpt, rpt)])
        _load_index_chunks(ei3_hbm, 0, isrc_v, wid, k, nchunks)
        _load_index_chunks(ei3_hbm, 1, idst_v, wid, k, nchunks)
        plsc.subcore_barrier()

        def gather(j, b):
            pltpu.async_copy(g_hbm.at[isrc_v.at[j]], rows_v.at[b], gsems.at[b])

        def gather_wait(j, b):
            pltpu.make_async_copy(g_hbm.at[isrc_v.at[j]], rows_v.at[b],
                                  gsems.at[b]).wait()

        def scat(j, b):
            pltpu.async_copy(rows_v.at[b], acc_sh.at[idst_v.at[j]], ssems.at[b],
                             add=True)

        def scat_wait(j, b):
            pltpu.make_async_copy(rows_v.at[b], acc_sh.at[idst_v.at[j]],
                                  ssems.at[b]).wait()

        for b in range(min(NB, k)):
            @pl.when(b < kw)
            def _():
                gather(b, b)

        def step(j, carry):
            b = lax.rem(j, NR)

            @pl.when(j < kw)
            def _():
                gather_wait(j, b)
                scat(j, b)
                jn = j + NB
                bn = lax.rem(jn, NR)

                @pl.when(jn < kw)
                def _():
                    @pl.when(jn >= NR)
                    def _():
                        scat_wait(jn - NR, bn)
                    gather(jn, bn)

            return carry

        lax.fori_loop(0, k, step, 0)

        def drain(j2, carry):
            jj = kw - NR + j2

            @pl.when(jj >= jnp.maximum(kw - NR, 0))
            def _():
                scat_wait(jj, lax.rem(jj, NR))

            return carry

        lax.fori_loop(jnp.maximum(NR - kw, 0), NR, drain, 0)
        plsc.subcore_barrier()
        pltpu.sync_copy(acc_sh.at[pl.ds(s * rpt, rpt)],
                        out_hbm.at[c, pl.ds(s * rpt, rpt)])

    return scat_kernel(g, ei3, zeros32).reshape(NC, nt // 4, CH)


def _dinv_packed(d_ref, rp):
    m = rp // 2
    d16 = d_ref[0] + d_ref[1]
    rows = lax.broadcasted_iota(jnp.int32, (rp, m), 0)
    cols = lax.broadcasted_iota(jnp.int32, (rp, m), 1)
    s_even = jnp.where((rows % 2 == 0) & (cols == rows // 2), 1.0, 0.0)
    s_odd = jnp.where((rows % 2 == 1) & (cols == rows // 2), 1.0, 0.0)
    a_e = jnp.dot(s_even, d16, preferred_element_type=jnp.float32)
    a_o = jnp.dot(s_odd, d16, preferred_element_type=jnp.float32)
    li = lax.broadcasted_iota(jnp.int32, (CH, CH), 0)
    lo = lax.broadcasted_iota(jnp.int32, (CH, CH), 1)
    r_e = jnp.where(li == 16 * (lo // 32), 1.0, 0.0)
    r_o = jnp.where(li == 64 + 16 * (lo // 32), 1.0, 0.0)
    d32 = (jnp.dot(a_e, r_e, preferred_element_type=jnp.float32)
           + jnp.dot(a_o, r_o, preferred_element_type=jnp.float32))
    return lax.rsqrt(d32 + 1.0)


def _tc_h(x4, W1s, nt, r):
    rp = r // 4

    def body(x_ref, w_ref, o_ref):
        o_ref[...] = jnp.dot(x_ref[...], w_ref[...],
                             preferred_element_type=jnp.float32)

    return pl.pallas_call(
        body,
        grid=(nt // r,),
        in_specs=[
            pl.BlockSpec((rp, x4.shape[1]), lambda i: (i, 0)),
            pl.BlockSpec(W1s.shape, lambda i: (0, 0)),
        ],
        out_specs=pl.BlockSpec((rp, CH), lambda i: (i, 0)),
        out_shape=jax.ShapeDtypeStruct((nt // 4, CH), jnp.float32),
    )(x4, W1s)


def _tc_scale(hp, degp, nt, r):
    rp = r // 4

    def body(h_ref, d_ref, o_ref):
        dinv = _dinv_packed(d_ref, rp)
        o_ref[...] = (h_ref[...] * dinv).astype(jnp.bfloat16)

    return pl.pallas_call(
        body,
        grid=(nt // r,),
        in_specs=[
            pl.BlockSpec((rp, CH), lambda i: (i, 0)),
            pl.BlockSpec((NC, rp // 2, CH), lambda i: (0, i, 0)),
        ],
        out_specs=pl.BlockSpec((rp, CH), lambda i: (i, 0)),
        out_shape=jax.ShapeDtypeStruct((nt // 4, CH), jnp.bfloat16),
    )(hp, degp)


def _tc_mid(sp, gp, degp, b1p, W2s, nt, r):
    rp = r // 4

    def body(s_ref, g_ref, d_ref, b_ref, w_ref, o_ref):
        dinv = _dinv_packed(d_ref, rp)
        stot = (s_ref[0] + s_ref[1] + g_ref[...]).astype(jnp.float32)
        z = jnp.maximum(stot * dinv + b_ref[...], 0.0)
        h = jnp.dot(z, w_ref[...], preferred_element_type=jnp.float32)
        o_ref[...] = (h * dinv).astype(jnp.bfloat16)

    return pl.pallas_call(
        body,
        grid=(nt // r,),
        in_specs=[
            pl.BlockSpec((NC, rp, CH), lambda i: (0, i, 0)),
            pl.BlockSpec((rp, CH), lambda i: (i, 0)),
            pl.BlockSpec((NC, rp // 2, CH), lambda i: (0, i, 0)),
            pl.BlockSpec(b1p.shape, lambda i: (0, 0)),
            pl.BlockSpec(W2s.shape, lambda i: (0, 0)),
        ],
        out_specs=pl.BlockSpec((rp, CH), lambda i: (i, 0)),
        out_shape=jax.ShapeDtypeStruct((nt // 4, CH), jnp.bfloat16),
    )(sp, gp, degp, b1p, W2s)


def _tc_head(sp, gp, degp, b2p, Wfs, bfp, nt, r):
    rp = r // 4

    def body(s_ref, g_ref, d_ref, b_ref, w_ref, bf_ref, o_ref):
        dinv = _dinv_packed(d_ref, rp)
        stot = (s_ref[0] + s_ref[1] + g_ref[...]).astype(jnp.float32)
        z = jnp.maximum(stot * dinv + b_ref[...], 0.0)
        h = jnp.dot(z, w_ref[...], preferred_element_type=jnp.float32)
        o_ref[...] = jax.nn.sigmoid(h + bf_ref[...])

    return pl.pallas_call(
        body,
        grid=(nt // r,),
        in_specs=[
            pl.BlockSpec((NC, rp, CH), lambda i: (0, i, 0)),
            pl.BlockSpec((rp, CH), lambda i: (i, 0)),
            pl.BlockSpec((NC, rp // 2, CH), lambda i: (0, i, 0)),
            pl.BlockSpec(b2p.shape, lambda i: (0, 0)),
            pl.BlockSpec(Wfs.shape, lambda i: (0, 0)),
            pl.BlockSpec(bfp.shape, lambda i: (0, 0)),
        ],
        out_specs=pl.BlockSpec((rp, Wfs.shape[1]), lambda i: (i, 0)),
        out_shape=jax.ShapeDtypeStruct((nt // 4, Wfs.shape[1]), jnp.float32),
    )(sp, gp, degp, b2p, Wfs, bfp)


def kernel(x, edge_index, W1, b1, W2, b2, Wfc, bfc):
    n, din = x.shape
    e = edge_index.shape[1]
    r = 2048
    nt = -(-n // r) * r
    dout = Wfc.shape[1]
    assert e % CH == 0 and nt % (8 * NS) == 0

    ei3 = edge_index.reshape(2, e // CH, CH)
    k = -(-(e // CH) // NW)

    x4 = jnp.pad(x, ((0, nt - n), (0, 0))).reshape(nt // 4, 4 * din)
    zeros16 = jnp.zeros((nt // NS, 16), jnp.float32)
    zerosb = jnp.zeros((nt // NS, DH), jnp.bfloat16)
    ones16 = jnp.ones((CH, 16), jnp.float32)

    eye4 = jnp.eye(4, dtype=jnp.float32)
    W1s = jnp.kron(eye4, W1)
    W2s = jnp.kron(eye4, W2)
    wfc_p = jnp.pad(Wfc, ((0, 0), (0, 8 - dout)))
    Wfs = jnp.kron(eye4, wfc_p)
    b1p = jnp.tile(b1, 4).reshape(1, CH)
    b2p = jnp.tile(b2, 4).reshape(1, CH)
    bfp = jnp.tile(jnp.pad(bfc, (0, 8 - dout)), 4).reshape(1, DH)

    degp = _sc_degree(ei3, zeros16, ones16, nt, k)
    hp = _tc_h(x4, W1s, nt, r)
    g1p = _tc_scale(hp, degp, nt, r)
    s1p = _sc_scatter(g1p, ei3, zerosb, nt, k)
    g2p = _tc_mid(s1p, g1p, degp, b1p, W2s, nt, r)
    s2p = _sc_scatter(g2p, ei3, zerosb, nt, k)
    outp = _tc_head(s2p, g2p, degp, b2p, Wfs, bfp, nt, r)

    return outp[:n // 4].reshape(n, 8)[:, :dout]


# speedup vs baseline: 1.1031x; 1.0682x over previous
import functools

import jax
import jax.numpy as jnp
from jax import lax
from jax.experimental import pallas as pl
from jax.experimental.pallas import tpu as pltpu
from jax.experimental.pallas import tpu_sc as plsc

NC = 2
NS = 16
NW = NC * NS
CH = 128
NB = 6
NR = 12
DH = 32


def _load_index_chunks(ei3_hbm, which, idx_v, wid, k, nchunks):
    last_full = nchunks // k
    klast = nchunks - last_full * k

    @pl.when(wid < last_full)
    def _():
        pltpu.sync_copy(ei3_hbm.at[which, pl.ds(wid * k, k)], idx_v)

    if klast > 0:
        @pl.when(wid == last_full)
        def _():
            pltpu.sync_copy(ei3_hbm.at[which, pl.ds(last_full * k, klast)],
                            idx_v.at[pl.ds(0, klast)])


def _sc_degree(ei3, zeros16, ones16, nt, k):
    nchunks = ei3.shape[1]
    rpt = nt // NS
    mesh = plsc.VectorSubcoreMesh(core_axis_name="c", subcore_axis_name="s")

    @functools.partial(
        pl.kernel,
        out_type=jax.ShapeDtypeStruct((NC, nt, 16), jnp.float32),
        mesh=mesh,
        scratch_types=[
            pltpu.VMEM((k, CH), jnp.int32),
            pltpu.VMEM((CH, 16), jnp.float32),
            pltpu.VMEM_SHARED((nt, 16), jnp.float32),
        ],
        compiler_params=pltpu.CompilerParams(use_tc_tiling_on_sc=False),
    )
    def deg_kernel(ei3_hbm, z_hbm, ones_hbm, out_hbm, idx_v, ones_v, acc_sh):
        c = lax.axis_index("c")
        s = lax.axis_index("s")
        wid = c * NS + s
        kw = jnp.clip(nchunks - wid * k, 0, k)
        pltpu.sync_copy(ones_hbm, ones_v)
        pltpu.sync_copy(z_hbm, acc_sh.at[pl.ds(s * rpt, rpt)])
        _load_index_chunks(ei3_hbm, 1, idx_v, wid, k, nchunks)
        plsc.subcore_barrier()

        def step(j, carry):
            @pl.when(j < kw)
            def _():
                pltpu.sync_copy(ones_v, acc_sh.at[idx_v.at[j]], add=True)

            return carry

        lax.fori_loop(0, k, step, 0)
        plsc.subcore_barrier()
        pltpu.sync_copy(acc_sh.at[pl.ds(s * rpt, rpt)],
                        out_hbm.at[c, pl.ds(s * rpt, rpt)])

    return deg_kernel(ei3, zeros16, ones16).reshape(NC, nt // 8, CH)


def _sc_scatter(gp, ei3, zeros32, nt, k):
    g = gp.reshape(nt, DH)
    nchunks = ei3.shape[1]
    rpt = nt // NS
    mesh = plsc.VectorSubcoreMesh(core_axis_name="c", subcore_axis_name="s")

    @functools.partial(
        pl.kernel,
        out_type=jax.ShapeDtypeStruct((NC, nt, DH), jnp.bfloat16),
        mesh=mesh,
        scratch_types=[
            pltpu.VMEM((k, CH), jnp.int32),
            pltpu.VMEM((k, CH), jnp.int32),
            pltpu.VMEM((NR, CH, DH), jnp.bfloat16),
            pltpu.VMEM_SHARED((nt, DH), jnp.bfloat16),
            pltpu.SemaphoreType.DMA((NR,)),
            pltpu.SemaphoreType.DMA((NR,)),
        ],
        compiler_params=pltpu.CompilerParams(use_tc_tiling_on_sc=False),
    )
    def scat_kernel(g_hbm, ei3_hbm, z_hbm, out_hbm,
                    isrc_v, idst_v, rows_v, acc_sh, gsems, ssems):
        c = lax.axis_index("c")
        s = lax.axis_index("s")
        wid = c * NS + s
        kw = jnp.clip(nchunks - wid * k, 0, k)
        pltpu.sync_copy(z_hbm, acc_sh.at[pl.ds(s * rpt, rpt)])
        _load_index_chunks(ei3_hbm, 0, isrc_v, wid, k, nchunks)
        _load_index_chunks(ei3_hbm, 1, idst_v, wid, k, nchunks)
        plsc.subcore_barrier()

        def gather(j, b):
            pltpu.async_copy(g_hbm.at[isrc_v.at[j]], rows_v.at[b], gsems.at[b])

        def gather_wait(j, b):
            pltpu.make_async_copy(g_hbm.at[isrc_v.at[j]], rows_v.at[b],
                                  gsems.at[b]).wait()

        def scat(j, b):
            pltpu.async_copy(rows_v.at[b], acc_sh.at[idst_v.at[j]], ssems.at[b],
                             add=True)

        def scat_wait(j, b):
            pltpu.make_async_copy(rows_v.at[b], acc_sh.at[idst_v.at[j]],
                                  ssems.at[b]).wait()

        for b in range(min(NB, k)):
            @pl.when(b < kw)
            def _():
                gather(b, b)

        def step(j, carry):
            b = lax.rem(j, NR)

            @pl.when(j < kw)
            def _():
                gather_wait(j, b)
                scat(j, b)
                jn = j + NB
                bn = lax.rem(jn, NR)

                @pl.when(jn < kw)
                def _():
                    @pl.when(jn >= NR)
                    def _():
                        scat_wait(jn - NR, bn)
                    gather(jn, bn)

            return carry

        lax.fori_loop(0, k, step, 0)

        def drain(j2, carry):
            jj = kw - NR + j2

            @pl.when(jj >= jnp.maximum(kw - NR, 0))
            def _():
                scat_wait(jj, lax.rem(jj, NR))

            return carry

        lax.fori_loop(jnp.maximum(NR - kw, 0), NR, drain, 0)
        plsc.subcore_barrier()
        pltpu.sync_copy(acc_sh.at[pl.ds(s * rpt, rpt)],
                        out_hbm.at[c, pl.ds(s * rpt, rpt)])

    return scat_kernel(g, ei3, zeros32).reshape(NC, nt // 4, CH)


def _dinv_packed(d_ref, rp):
    m = rp // 2
    d16 = d_ref[0] + d_ref[1]
    rows = lax.broadcasted_iota(jnp.int32, (rp, m), 0)
    cols = lax.broadcasted_iota(jnp.int32, (rp, m), 1)
    s_even = jnp.where((rows % 2 == 0) & (cols == rows // 2), 1.0, 0.0)
    s_odd = jnp.where((rows % 2 == 1) & (cols == rows // 2), 1.0, 0.0)
    a_e = jnp.dot(s_even, d16, preferred_element_type=jnp.float32)
    a_o = jnp.dot(s_odd, d16, preferred_element_type=jnp.float32)
    li = lax.broadcasted_iota(jnp.int32, (CH, CH), 0)
    lo = lax.broadcasted_iota(jnp.int32, (CH, CH), 1)
    r_e = jnp.where(li == 16 * (lo // 32), 1.0, 0.0)
    r_o = jnp.where(li == 64 + 16 * (lo // 32), 1.0, 0.0)
    d32 = (jnp.dot(a_e, r_e, preferred_element_type=jnp.float32)
           + jnp.dot(a_o, r_o, preferred_element_type=jnp.float32))
    return lax.rsqrt(d32 + 1.0)


def _tc_h(x4, W1s, nt, r):
    rp = r // 4

    def body(x_ref, w_ref, o_ref):
        o_ref[...] = jnp.dot(x_ref[...], w_ref[...],
                             preferred_element_type=jnp.float32)

    return pl.pallas_call(
        body,
        grid=(nt // r,),
        in_specs=[
            pl.BlockSpec((rp, x4.shape[1]), lambda i: (i, 0)),
            pl.BlockSpec(W1s.shape, lambda i: (0, 0)),
        ],
        out_specs=pl.BlockSpec((rp, CH), lambda i: (i, 0)),
        out_shape=jax.ShapeDtypeStruct((nt // 4, CH), jnp.float32),
    )(x4, W1s)


def _tc_scale(hp, degp, nt, r):
    rp = r // 4

    def body(h_ref, d_ref, o_ref):
        dinv = _dinv_packed(d_ref, rp)
        o_ref[...] = (h_ref[...] * dinv).astype(jnp.bfloat16)

    return pl.pallas_call(
        body,
        grid=(nt // r,),
        in_specs=[
            pl.BlockSpec((rp, CH), lambda i: (i, 0)),
            pl.BlockSpec((NC, rp // 2, CH), lambda i: (0, i, 0)),
        ],
        out_specs=pl.BlockSpec((rp, CH), lambda i: (i, 0)),
        out_shape=jax.ShapeDtypeStruct((nt // 4, CH), jnp.bfloat16),
    )(hp, degp)


def _tc_mid(sp, gp, degp, b1p, W2s, nt, r):
    rp = r // 4

    def body(s_ref, g_ref, d_ref, b_ref, w_ref, o_ref):
        dinv = _dinv_packed(d_ref, rp)
        stot = (s_ref[0] + s_ref[1] + g_ref[...]).astype(jnp.float32)
        z = jnp.maximum(stot * dinv + b_ref[...], 0.0)
        h = jnp.dot(z, w_ref[...], preferred_element_type=jnp.float32)
        o_ref[...] = (h * dinv).astype(jnp.bfloat16)

    return pl.pallas_call(
        body,
        grid=(nt // r,),
        in_specs=[
            pl.BlockSpec((NC, rp, CH), lambda i: (0, i, 0)),
            pl.BlockSpec((rp, CH), lambda i: (i, 0)),
            pl.BlockSpec((NC, rp // 2, CH), lambda i: (0, i, 0)),
            pl.BlockSpec(b1p.shape, lambda i: (0, 0)),
            pl.BlockSpec(W2s.shape, lambda i: (0, 0)),
        ],
        out_specs=pl.BlockSpec((rp, CH), lambda i: (i, 0)),
        out_shape=jax.ShapeDtypeStruct((nt // 4, CH), jnp.bfloat16),
    )(sp, gp, degp, b1p, W2s)


def _tc_head(sp, gp, degp, b2p, Wfs, bfp, nt, r):
    rp = r // 4

    def body(s_ref, g_ref, d_ref, b_ref, w_ref, bf_ref, o_ref):
        dinv = _dinv_packed(d_ref, rp)
        stot = (s_ref[0] + s_ref[1] + g_ref[...]).astype(jnp.float32)
        z = jnp.maximum(stot * dinv + b_ref[...], 0.0)
        h = jnp.dot(z, w_ref[...], preferred_element_type=jnp.float32)
        o_ref[...] = jax.nn.sigmoid(h + bf_ref[...])

    return pl.pallas_call(
        body,
        grid=(nt // r,),
        in_specs=[
            pl.BlockSpec((NC, rp, CH), lambda i: (0, i, 0)),
            pl.BlockSpec((rp, CH), lambda i: (i, 0)),
            pl.BlockSpec((NC, rp // 2, CH), lambda i: (0, i, 0)),
            pl.BlockSpec(b2p.shape, lambda i: (0, 0)),
            pl.BlockSpec(Wfs.shape, lambda i: (0, 0)),
            pl.BlockSpec(bfp.shape, lambda i: (0, 0)),
        ],
        out_specs=pl.BlockSpec((rp, Wfs.shape[1]), lambda i: (i, 0)),
        out_shape=jax.ShapeDtypeStruct((nt // 4, Wfs.shape[1]), jnp.float32),
    )(sp, gp, degp, b2p, Wfs, bfp)


def kernel(x, edge_index, W1, b1, W2, b2, Wfc, bfc):
    n, din = x.shape
    e = edge_index.shape[1]
    r = 2048
    nt = -(-n // r) * r
    dout = Wfc.shape[1]
    assert e % CH == 0 and nt % (8 * NS) == 0

    ei3 = edge_index.reshape(2, e // CH, CH)
    k = -(-(e // CH) // NW)

    x4 = jnp.pad(x, ((0, nt - n), (0, 0))).reshape(nt // 4, 4 * din)
    zeros16 = jnp.zeros((nt // NS, 16), jnp.float32)
    zerosb = jnp.zeros((nt // NS, DH), jnp.bfloat16)
    ones16 = jnp.ones((CH, 16), jnp.float32)

    eye4 = jnp.eye(4, dtype=jnp.float32)
    W1s = jnp.kron(eye4, W1)
    W2s = jnp.kron(eye4, W2)
    wfc_p = jnp.pad(Wfc, ((0, 0), (0, 8 - dout)))
    Wfs = jnp.kron(eye4, wfc_p)
    b1p = jnp.tile(b1, 4).reshape(1, CH)
    b2p = jnp.tile(b2, 4).reshape(1, CH)
    bfp = jnp.tile(jnp.pad(bfc, (0, 8 - dout)), 4).reshape(1, DH)

    degp = _sc_degree(ei3, zeros16, ones16, nt, k)
    hp = _tc_h(x4, W1s, nt, r)
    g1p = _tc_scale(hp, degp, nt, r)
    s1p = _sc_scatter(g1p, ei3, zerosb, nt, k)
    g2p = _tc_mid(s1p, g1p, degp, b1p, W2s, nt, r)
    s2p = _sc_scatter(g2p, ei3, zerosb, nt, k)
    outp = _tc_head(s2p, g2p, degp, b2p, Wfs, bfp, nt, r)

    return outp[:n // 4].reshape(n, 8)[:, :dout]
